# Initial kernel scaffold; baseline (speedup 1.0000x reference)
import jax
import jax.numpy as jnp
from jax.experimental import pallas as pl


def kernel(x, edge_index, face, node_weights, batch, v):
    raise NotImplementedError("write your pallas kernel here")


# speedup vs baseline: 46.6119x; 46.6119x over previous
import functools

import jax
import jax.numpy as jnp
from jax import lax
from jax.experimental import pallas as pl
from jax.experimental.pallas import tpu as pltpu
from jax.experimental.pallas import tpu_sc as plsc

N = 10000
E = 160000
F = 20000
D = 3
T = 16
S = 16
R = 1.1
B = 8

DELTA = 2.0 * R / (S - 1)
NW = 32
N_PAD = 10240
E_PAD = 163840
F_PAD = 20480
E_CHUNK = 1024
E_CHUNKS = 5
F_CHUNK = 640
N_CHUNK = 320


def _pack_body(x_ref, v_ref, w_ref, b_ref, out_ref):
    xv = x_ref[...]
    vv = v_ref[...]
    nh = (xv[:, 0:1] * vv[0:1, :]
          + xv[:, 1:2] * vv[1:2, :]
          + xv[:, 2:3] * vv[2:3, :])
    out_ref[:, 0:16] = nh
    out_ref[:, 16:17] = w_ref[...]
    out_ref[:, 17:18] = b_ref[...]
    out_ref[:, 18:32] = jnp.zeros((N_PAD, 14), jnp.float32)


def _fin_body(hist_ref, out_ref):
    s = jnp.sum(hist_ref[...], axis=0)
    h2 = s[0:128, :]
    g2 = s[128:256, :]
    r = lax.broadcasted_iota(jnp.int32, (128, 128), 0)
    c = lax.broadcasted_iota(jnp.int32, (128, 128), 1)
    m = ((r >> 4) == (c >> 4)) & ((c & 15) < (r & 15))
    out_ref[...] = h2 + jnp.dot(m.astype(jnp.float32), g2,
                                preferred_element_type=jnp.float32)


def _sc_body(packed_hbm, ei0_hbm, ei1_hbm, fa0_hbm, fa1_hbm, fa2_hbm,
             out_hbm, hist_v, stage_v, r0_v, r1_v, r2_v, i0_v, i1_v,
             f0_v, f1_v, f2_v, sem):
    cid = lax.axis_index("c")
    sid = lax.axis_index("s")
    wid = sid * 2 + cid
    lane = lax.iota(jnp.int32, 16)

    zero16 = jnp.zeros((16,), jnp.float32)

    def _zero(i, carry):
        hist_v[pl.ds(i * 16, 16)] = zero16
        return carry

    lax.fori_loop(0, 256, _zero, 0)

    inv = 1.0 / DELTA
    c0 = R / DELTA + 0.5
    scale = -500.0 * DELTA

    def _accum(h, sw, bv):
        u = h * inv + c0
        jf = jnp.minimum(jnp.maximum(u, 0.0), 15.0)
        j = jf.astype(jnp.int32)
        jq = j.astype(jnp.float32)
        z = jq * scale + (h * 500.0 + 500.0 * R)
        z = jnp.minimum(jnp.maximum(z, -30.0), 30.0)
        wsig = sw / (1.0 + jnp.exp(z))
        idx = (bv.astype(jnp.int32) * 16 + j) * 16 + lane
        plsc.addupdate_scatter(hist_v, [idx], wsig)
        plsc.addupdate_scatter(hist_v, [idx + 2048], sw)

    for ch in range(E_CHUNKS):
        blk = wid * E_CHUNKS + ch
        pltpu.sync_copy(ei0_hbm.at[blk], i0_v)
        pltpu.sync_copy(ei1_hbm.at[blk], i1_v)
        cps = []
        for a in range(8):
            cps.append(pltpu.async_copy(
                packed_hbm.at[i0_v.at[a]], r0_v.at[pl.ds(a * 128, 128)], sem))
            cps.append(pltpu.async_copy(
                packed_hbm.at[i1_v.at[a]], r1_v.at[pl.ds(a * 128, 128)], sem))
        for cp in cps:
            cp.wait()

        def _ebody(e, carry):
            h = jnp.minimum(r0_v[e, 0:16], r1_v[e, 0:16])
            s0 = r0_v[e, 16:32]
            s1 = r1_v[e, 16:32]
            wm = jnp.maximum(s0, s1)
            zi = jnp.zeros((16,), jnp.int32)
            wv = wm.at[zi].get(mode="promise_in_bounds")
            bv = s0.at[zi + 1].get(mode="promise_in_bounds")
            _accum(h, -wv, bv)
            return carry

        lax.fori_loop(0, E_CHUNK, _ebody, 0)

    pltpu.sync_copy(fa0_hbm.at[wid], f0_v)
    pltpu.sync_copy(fa1_hbm.at[wid], f1_v)
    pltpu.sync_copy(fa2_hbm.at[wid], f2_v)
    cps = []
    for a in range(5):
        cps.append(pltpu.async_copy(
            packed_hbm.at[f0_v.at[a]], r0_v.at[pl.ds(a * 128, 128)], sem))
        cps.append(pltpu.async_copy(
            packed_hbm.at[f1_v.at[a]], r1_v.at[pl.ds(a * 128, 128)], sem))
        cps.append(pltpu.async_copy(
            packed_hbm.at[f2_v.at[a]], r2_v.at[pl.ds(a * 128, 128)], sem))
    for cp in cps:
        cp.wait()

    def _fbody(e, carry):
        h = jnp.minimum(jnp.minimum(r0_v[e, 0:16], r1_v[e, 0:16]),
                        r2_v[e, 0:16])
        s0 = r0_v[e, 16:32]
        wm = jnp.maximum(jnp.maximum(s0, r1_v[e, 16:32]), r2_v[e, 16:32])
        zi = jnp.zeros((16,), jnp.int32)
        wv = wm.at[zi].get(mode="promise_in_bounds")
        bv = s0.at[zi + 1].get(mode="promise_in_bounds")
        _accum(h, wv, bv)
        return carry

    lax.fori_loop(0, F_CHUNK, _fbody, 0)

    nbase = pl.multiple_of(wid * N_CHUNK, N_CHUNK)
    pltpu.sync_copy(packed_hbm.at[pl.ds(nbase, N_CHUNK)],
                    r0_v.at[pl.ds(0, N_CHUNK)])

    def _nbody(e, carry):
        h = r0_v[e, 0:16]
        s0 = r0_v[e, 16:32]
        zi = jnp.zeros((16,), jnp.int32)
        wv = s0.at[zi].get(mode="promise_in_bounds")
        bv = s0.at[zi + 1].get(mode="promise_in_bounds")
        _accum(h, wv, bv)
        return carry

    lax.fori_loop(0, N_CHUNK, _nbody, 0)

    def _stage(i, carry):
        stage_v[i, :] = hist_v[pl.ds(i * 16, 16)]
        return carry

    lax.fori_loop(0, 256, _stage, 0)
    pltpu.sync_copy(stage_v, out_hbm.at[wid])


_sc_call = pl.kernel(
    _sc_body,
    out_type=jax.ShapeDtypeStruct((NW, 256, 16), jnp.float32),
    mesh=plsc.VectorSubcoreMesh(core_axis_name="c", subcore_axis_name="s"),
    compiler_params=pltpu.CompilerParams(needs_layout_passes=False,
                                         use_tc_tiling_on_sc=False),
    scratch_types=[
        pltpu.VMEM((4096,), jnp.float32),
        pltpu.VMEM((256, 16), jnp.float32),
        pltpu.VMEM((E_CHUNK, 32), jnp.float32),
        pltpu.VMEM((E_CHUNK, 32), jnp.float32),
        pltpu.VMEM((E_CHUNK, 32), jnp.float32),
        pltpu.VMEM((8, 128), jnp.int32),
        pltpu.VMEM((8, 128), jnp.int32),
        pltpu.VMEM((5, 128), jnp.int32),
        pltpu.VMEM((5, 128), jnp.int32),
        pltpu.VMEM((5, 128), jnp.int32),
        pltpu.SemaphoreType.DMA,
    ],
)


@jax.jit
def kernel(x, edge_index, face, node_weights, batch, v):
    ei = edge_index.astype(jnp.int32)
    fa = face.astype(jnp.int32)

    xp = jnp.concatenate([x, jnp.zeros((N_PAD - N, D), jnp.float32)], axis=0)
    wp = jnp.concatenate([node_weights,
                          jnp.zeros((N_PAD - N,), jnp.float32)])[:, None]
    bp = jnp.concatenate([batch.astype(jnp.float32),
                          jnp.zeros((N_PAD - N,), jnp.float32)])[:, None]

    packed = pl.pallas_call(
        _pack_body,
        out_shape=jax.ShapeDtypeStruct((N_PAD, 32), jnp.float32),
    )(xp, v, wp, bp)

    epad = jnp.full((E_PAD - E,), N, jnp.int32)
    fpad = jnp.full((F_PAD - F,), N, jnp.int32)
    ei0 = jnp.concatenate([ei[0], epad]).reshape(NW * E_CHUNKS, 8, 128)
    ei1 = jnp.concatenate([ei[1], epad]).reshape(NW * E_CHUNKS, 8, 128)
    fa0 = jnp.concatenate([fa[0], fpad]).reshape(NW, 5, 128)
    fa1 = jnp.concatenate([fa[1], fpad]).reshape(NW, 5, 128)
    fa2 = jnp.concatenate([fa[2], fpad]).reshape(NW, 5, 128)

    hist = _sc_call(packed, ei0, ei1, fa0, fa1, fa2)

    out2 = pl.pallas_call(
        _fin_body,
        out_shape=jax.ShapeDtypeStruct((128, 16), jnp.float32),
    )(hist)
    return out2.reshape(B, S, T)


# speedup vs baseline: 73.9675x; 1.5869x over previous
import functools

import jax
import jax.numpy as jnp
from jax import lax
from jax.experimental import pallas as pl
from jax.experimental.pallas import tpu as pltpu
from jax.experimental.pallas import tpu_sc as plsc

N = 10000
E = 160000
F = 20000
D = 3
T = 16
S = 16
R = 1.1
B = 8

DELTA = 2.0 * R / (S - 1)
NW = 32
N_PAD = 10240
E_PAD = 163840
F_PAD = 20480
E_CHUNK = 1024
E_CHUNKS = 5
F_CHUNK = 640
N_CHUNK = 320


def _pack_body(x_ref, v_ref, w_ref, b_ref, out_ref):
    xv = x_ref[...]
    vv = v_ref[...]
    nh = (xv[:, 0:1] * vv[0:1, :]
          + xv[:, 1:2] * vv[1:2, :]
          + xv[:, 2:3] * vv[2:3, :])
    out_ref[:, 0:16] = nh
    out_ref[:, 16:17] = w_ref[...]
    out_ref[:, 17:18] = b_ref[...]
    out_ref[:, 18:32] = jnp.zeros((N_PAD, 14), jnp.float32)


def _fin_body(hist_ref, out_ref):
    s = jnp.sum(hist_ref[...], axis=0)
    h2 = s[0:128, :]
    g2 = s[128:256, :]
    r = lax.broadcasted_iota(jnp.int32, (128, 128), 0)
    c = lax.broadcasted_iota(jnp.int32, (128, 128), 1)
    m = ((r >> 4) == (c >> 4)) & ((c & 15) < (r & 15))
    out_ref[...] = h2 + jnp.dot(m.astype(jnp.float32), g2,
                                preferred_element_type=jnp.float32)


def _sc_body(packed_hbm, ei0_hbm, ei1_hbm, fa0_hbm, fa1_hbm, fa2_hbm,
             out_hbm, hist_v, stage_v, r0_v, r1_v, r2_v, i0_v, i1_v,
             f0_v, f1_v, f2_v, sem):
    cid = lax.axis_index("c")
    sid = lax.axis_index("s")
    wid = sid * 2 + cid
    lane = lax.iota(jnp.int32, 16)

    zero16 = jnp.zeros((16,), jnp.float32)

    def _zero(i, carry):
        hist_v[pl.ds(i * 16, 16)] = zero16
        return carry

    lax.fori_loop(0, 256, _zero, 0)

    inv = 1.0 / DELTA
    c0 = R / DELTA + 0.5
    scale = -500.0 * DELTA

    def _accum(h, sw, bv):
        u = h * inv + c0
        jf = jnp.minimum(jnp.maximum(u, 0.0), 15.0)
        j = jf.astype(jnp.int32)
        jq = j.astype(jnp.float32)
        z = jq * scale + (h * 500.0 + 500.0 * R)
        z = jnp.minimum(jnp.maximum(z, -30.0), 30.0)
        wsig = sw / (1.0 + jnp.exp(z))
        idx = bv.astype(jnp.int32) + j * 16 + lane
        plsc.addupdate_scatter(hist_v, [idx], wsig)
        plsc.addupdate_scatter(hist_v, [idx + 2048], sw)

    for ch in range(E_CHUNKS):
        blk = wid * E_CHUNKS + ch
        pltpu.sync_copy(ei0_hbm.at[blk], i0_v)
        pltpu.sync_copy(ei1_hbm.at[blk], i1_v)
        cps = []
        for a in range(8):
            cps.append(pltpu.async_copy(
                packed_hbm.at[i0_v.at[a]], r0_v.at[pl.ds(a * 128, 128)], sem))
            cps.append(pltpu.async_copy(
                packed_hbm.at[i1_v.at[a]], r1_v.at[pl.ds(a * 128, 128)], sem))
        for cp in cps:
            cp.wait()

        @plsc.parallel_loop(0, E_CHUNK, 1, unroll=4)
        def _ebody(e):
            h = jnp.minimum(r0_v[e, 0:16], r1_v[e, 0:16])
            s0 = r0_v[e, 16:32]
            s1 = r1_v[e, 16:32]
            wm = jnp.maximum(s0, s1)
            zi = jnp.zeros((16,), jnp.int32)
            wv = wm.at[zi].get(mode="promise_in_bounds")
            bv = s0.at[zi + 1].get(mode="promise_in_bounds")
            _accum(h, -wv, bv)

    pltpu.sync_copy(fa0_hbm.at[wid], f0_v)
    pltpu.sync_copy(fa1_hbm.at[wid], f1_v)
    pltpu.sync_copy(fa2_hbm.at[wid], f2_v)
    cps = []
    for a in range(5):
        cps.append(pltpu.async_copy(
            packed_hbm.at[f0_v.at[a]], r0_v.at[pl.ds(a * 128, 128)], sem))
        cps.append(pltpu.async_copy(
            packed_hbm.at[f1_v.at[a]], r1_v.at[pl.ds(a * 128, 128)], sem))
        cps.append(pltpu.async_copy(
            packed_hbm.at[f2_v.at[a]], r2_v.at[pl.ds(a * 128, 128)], sem))
    for cp in cps:
        cp.wait()

    @plsc.parallel_loop(0, F_CHUNK, 1, unroll=4)
    def _fbody(e):
        h = jnp.minimum(jnp.minimum(r0_v[e, 0:16], r1_v[e, 0:16]),
                        r2_v[e, 0:16])
        s0 = r0_v[e, 16:32]
        wm = jnp.maximum(jnp.maximum(s0, r1_v[e, 16:32]), r2_v[e, 16:32])
        zi = jnp.zeros((16,), jnp.int32)
        wv = wm.at[zi].get(mode="promise_in_bounds")
        bv = s0.at[zi + 1].get(mode="promise_in_bounds")
        _accum(h, wv, bv)

    nbase = pl.multiple_of(wid * N_CHUNK, N_CHUNK)
    pltpu.sync_copy(packed_hbm.at[pl.ds(nbase, N_CHUNK)],
                    r0_v.at[pl.ds(0, N_CHUNK)])

    @plsc.parallel_loop(0, N_CHUNK, 1, unroll=4)
    def _nbody(e):
        h = r0_v[e, 0:16]
        s0 = r0_v[e, 16:32]
        zi = jnp.zeros((16,), jnp.int32)
        wv = s0.at[zi].get(mode="promise_in_bounds")
        bv = s0.at[zi + 1].get(mode="promise_in_bounds")
        _accum(h, wv, bv)

    def _stage(i, carry):
        stage_v[i, :] = hist_v[pl.ds(i * 16, 16)]
        return carry

    lax.fori_loop(0, 256, _stage, 0)
    pltpu.sync_copy(stage_v, out_hbm.at[wid])


_sc_call = pl.kernel(
    _sc_body,
    out_type=jax.ShapeDtypeStruct((NW, 256, 16), jnp.float32),
    mesh=plsc.VectorSubcoreMesh(core_axis_name="c", subcore_axis_name="s"),
    compiler_params=pltpu.CompilerParams(needs_layout_passes=False,
                                         use_tc_tiling_on_sc=False),
    scratch_types=[
        pltpu.VMEM((4096,), jnp.float32),
        pltpu.VMEM((256, 16), jnp.float32),
        pltpu.VMEM((E_CHUNK, 32), jnp.float32),
        pltpu.VMEM((E_CHUNK, 32), jnp.float32),
        pltpu.VMEM((E_CHUNK, 32), jnp.float32),
        pltpu.VMEM((8, 128), jnp.int32),
        pltpu.VMEM((8, 128), jnp.int32),
        pltpu.VMEM((5, 128), jnp.int32),
        pltpu.VMEM((5, 128), jnp.int32),
        pltpu.VMEM((5, 128), jnp.int32),
        pltpu.SemaphoreType.DMA,
    ],
)


@jax.jit
def kernel(x, edge_index, face, node_weights, batch, v):
    ei = edge_index.astype(jnp.int32)
    fa = face.astype(jnp.int32)

    xp = jnp.concatenate([x, jnp.zeros((N_PAD - N, D), jnp.float32)], axis=0)
    wp = jnp.concatenate([node_weights,
                          jnp.zeros((N_PAD - N,), jnp.float32)])[:, None]
    bp = jnp.concatenate([batch.astype(jnp.float32) * 256.0,
                          jnp.zeros((N_PAD - N,), jnp.float32)])[:, None]

    packed = pl.pallas_call(
        _pack_body,
        out_shape=jax.ShapeDtypeStruct((N_PAD, 32), jnp.float32),
    )(xp, v, wp, bp)

    epad = jnp.full((E_PAD - E,), N, jnp.int32)
    fpad = jnp.full((F_PAD - F,), N, jnp.int32)
    ei0 = jnp.concatenate([ei[0], epad]).reshape(NW * E_CHUNKS, 8, 128)
    ei1 = jnp.concatenate([ei[1], epad]).reshape(NW * E_CHUNKS, 8, 128)
    fa0 = jnp.concatenate([fa[0], fpad]).reshape(NW, 5, 128)
    fa1 = jnp.concatenate([fa[1], fpad]).reshape(NW, 5, 128)
    fa2 = jnp.concatenate([fa[2], fpad]).reshape(NW, 5, 128)

    hist = _sc_call(packed, ei0, ei1, fa0, fa1, fa2)

    out2 = pl.pallas_call(
        _fin_body,
        out_shape=jax.ShapeDtypeStruct((128, 16), jnp.float32),
    )(hist)
    return out2.reshape(B, S, T)
